# baseline (speedup 1.0000x reference)
import jax
import jax.numpy as jnp
from jax.experimental import pallas as pl
from jax.experimental.pallas import tpu as pltpu

_N = 256
_GCN_OUT = 10
_NB = 2
_TILE = 12288


def _fwd_t_kernel(x_ref, weff_ref, cls_ref, o_ref):
    y = jax.lax.dot_general(
        weff_ref[...], x_ref[...], (((0,), (0,)), ((), ())),
        preferred_element_type=jnp.float32)
    y = jnp.maximum(y, 0.0)
    ones = jnp.ones((1, y.shape[1]), jnp.float32)
    y1 = jnp.concatenate([y, ones], axis=0)
    o_ref[...] = jax.lax.dot_general(
        cls_ref[...], y1, (((0,), (0,)), ((), ())),
        preferred_element_type=jnp.float32)


@jax.jit
def kernel(x, w_eff, cls_packed):
    bsz = x.shape[0]
    xt = x.reshape(bsz, _N).T
    tile = _TILE
    while bsz % tile:
        tile //= 2
    out_t = pl.pallas_call(
        _fwd_t_kernel,
        out_shape=jax.ShapeDtypeStruct((_NB, bsz), jnp.float32),
        grid=(bsz // tile,),
        in_specs=[
            pl.BlockSpec((_N, tile), lambda i: (0, i)),
            pl.BlockSpec((_N, _GCN_OUT), lambda i: (0, 0)),
            pl.BlockSpec((_GCN_OUT + 1, _NB), lambda i: (0, 0)),
        ],
        out_specs=pl.BlockSpec((_NB, tile), lambda i: (0, i)),
        compiler_params=pltpu.CompilerParams(
            dimension_semantics=("arbitrary",)),
    )(xt, w_eff, cls_packed)
    return out_t.T


# speedup vs baseline: 1.3370x; 1.3370x over previous
import jax
import jax.numpy as jnp
from jax.experimental import pallas as pl
from jax.experimental.pallas import tpu as pltpu

_N = 256
_GCN_OUT = 10
_NB = 2
_TILE = 8192


def _fwd_t_kernel(x_ref, weff_ref, cls_ref, o_ref):
    y = jnp.dot(weff_ref[...], x_ref[...],
                preferred_element_type=jnp.float32)
    y = jnp.maximum(y, 0.0)
    ones = jnp.ones((1, y.shape[1]), jnp.float32)
    y1 = jnp.concatenate([y, ones], axis=0)
    o_ref[...] = jnp.dot(cls_ref[...], y1,
                         preferred_element_type=jnp.float32)


@jax.jit
def kernel(x, w_eff, cls_packed):
    bsz = x.shape[0]
    xt = x.reshape(bsz, _N).T
    tile = _TILE
    while bsz % tile:
        tile //= 2
    out_t = pl.pallas_call(
        _fwd_t_kernel,
        out_shape=jax.ShapeDtypeStruct((_NB, bsz), jnp.float32),
        grid=(bsz // tile,),
        in_specs=[
            pl.BlockSpec((_N, tile), lambda i: (0, i)),
            pl.BlockSpec((_GCN_OUT, _N), lambda i: (0, 0)),
            pl.BlockSpec((_NB, _GCN_OUT + 1), lambda i: (0, 0)),
        ],
        out_specs=pl.BlockSpec((_NB, tile), lambda i: (0, i)),
        compiler_params=pltpu.CompilerParams(
            dimension_semantics=("arbitrary",)),
    )(xt, w_eff.T, cls_packed.T)
    return out_t.T
